# Initial kernel scaffold; baseline (speedup 1.0000x reference)
import jax
import jax.numpy as jnp
from jax.experimental import pallas as pl


def kernel(ranks, rerank_dba_final, res_top1000_dba, ranks_trans_1000_pre, x_dba):
    raise NotImplementedError("write your pallas kernel here")


# speedup vs baseline: 4.1404x; 4.1404x over previous
import functools

import jax
import jax.numpy as jnp
from jax import lax
from jax.experimental import pallas as pl
from jax.experimental.pallas import tpu as pltpu

_M = 400
_K = 10
_Q = 256
_N_DB = 100000
_D = 512

_HI = lax.Precision.HIGHEST


def _stable_desc_rank(v_row, iota_lane, iota_sub, ident):
    m = v_row.shape[1]
    v_col = lax.dot_general(ident, v_row, (((1,), (1,)), ((), ())),
                            precision=_HI)
    r_mat = jnp.broadcast_to(v_row, (m, m))
    l_mat = jnp.broadcast_to(v_col, (m, m))
    gt = r_mat > l_mat
    tie = (r_mat == l_mat) & (iota_lane < iota_sub)
    return jnp.sum((gt | tie).astype(jnp.float32), axis=1, keepdims=True)


def _compute_body(res_ref, perm_ref, val_ref, idx_ref, x_ref, out_ref):
    m = _M
    iota_lane = lax.broadcasted_iota(jnp.int32, (m, m), 1)
    iota_sub = lax.broadcasted_iota(jnp.int32, (m, m), 0)
    ident = (iota_lane == iota_sub).astype(jnp.float32)
    iota_lane_f = iota_lane.astype(jnp.float32)

    v_row = res_ref[0]
    rank1 = _stable_desc_rank(v_row, iota_lane, iota_sub, ident)
    oh1 = (jnp.broadcast_to(rank1, (m, m)) == iota_lane_f).astype(jnp.float32)
    sv_row = lax.dot_general(v_row, oh1, (((1,), (0,)), ((), ())),
                             precision=_HI)

    x_mat = x_ref[0]
    vmax = x_ref[0, pl.ds(idx_ref[0, 0, 0], 1), :]
    for k in range(1, _K):
        vmax = jnp.maximum(vmax, x_ref[0, pl.ds(idx_ref[0, 0, k], 1), :])

    s_row = lax.dot_general(vmax, x_mat, (((1,), (1,)), ((), ())),
                            precision=lax.Precision.DEFAULT)

    perm_row = perm_ref[0]
    p_mat = (iota_sub == jnp.broadcast_to(perm_row, (m, m))).astype(jnp.float32)
    s_perm = lax.dot_general(s_row, p_mat, (((1,), (0,)), ((), ())),
                             precision=_HI)
    r_row = (sv_row + s_perm) * 0.5
    rank2 = _stable_desc_rank(r_row, iota_lane, iota_sub, ident)

    oh2 = (jnp.broadcast_to(rank2, (m, m)) == iota_lane_f).astype(jnp.float32)
    val_row = val_ref[0].astype(jnp.float32)
    reord = lax.dot_general(val_row, oh2, (((1,), (0,)), ((), ())),
                            precision=_HI)
    out_ref[0] = reord.astype(jnp.int32)


def _assembly_body(reord_ref, ranks_ref, out_ref, *, rows):
    i = pl.program_id(0)

    @pl.when(i == 0)
    def _():
        rf = reord_ref[...].astype(jnp.float32)
        il = lax.broadcasted_iota(jnp.int32, (_Q, _Q), 1)
        isub = lax.broadcasted_iota(jnp.int32, (_Q, _Q), 0)
        ident_q = (il == isub).astype(jnp.float32)
        t = lax.dot_general(rf, ident_q, (((0,), (0,)), ((), ())),
                            precision=_HI)
        out_ref[0:_M, :] = t.astype(jnp.int32)
        out_ref[_M:rows, :] = ranks_ref[_M:rows, :]

    @pl.when(i != 0)
    def _():
        out_ref[...] = ranks_ref[...]


def kernel(ranks, rerank_dba_final, res_top1000_dba, ranks_trans_1000_pre, x_dba):
    q, m, d = _Q, _M, _D

    res3 = res_top1000_dba.reshape(q, 1, m)
    perm3 = ranks_trans_1000_pre.reshape(q, 1, m)
    val3 = rerank_dba_final.reshape(q, 1, m)
    idx_top = ranks_trans_1000_pre[:, :_K].reshape(q, 1, _K)

    reordered = pl.pallas_call(
        _compute_body,
        grid=(q,),
        in_specs=[
            pl.BlockSpec((1, 1, m), lambda i: (i, 0, 0)),
            pl.BlockSpec((1, 1, m), lambda i: (i, 0, 0)),
            pl.BlockSpec((1, 1, m), lambda i: (i, 0, 0)),
            pl.BlockSpec((1, 1, _K), lambda i: (i, 0, 0), memory_space=pltpu.SMEM),
            pl.BlockSpec((1, m, d), lambda i: (i, 0, 0)),
        ],
        out_specs=pl.BlockSpec((1, 1, m), lambda i: (i, 0, 0)),
        out_shape=jax.ShapeDtypeStruct((q, 1, m), jnp.int32),
    )(res3, perm3, val3, idx_top, x_dba)

    rows = 4000
    nblk = _N_DB // rows
    out = pl.pallas_call(
        functools.partial(_assembly_body, rows=rows),
        grid=(nblk,),
        in_specs=[
            pl.BlockSpec((q, m), lambda i: (0, 0)),
            pl.BlockSpec((rows, q), lambda i: (i, 0)),
        ],
        out_specs=pl.BlockSpec((rows, q), lambda i: (i, 0)),
        out_shape=jax.ShapeDtypeStruct((_N_DB, q), jnp.int32),
    )(reordered.reshape(q, m), ranks)
    return out


# speedup vs baseline: 6.9619x; 1.6815x over previous
import functools

import jax
import jax.numpy as jnp
from jax import lax
from jax.experimental import pallas as pl
from jax.experimental.pallas import tpu as pltpu

_M = 400
_K = 10
_Q = 256
_N_DB = 100000
_D = 512

_HI = lax.Precision.HIGHEST


def _to_col(row, ident):
    return lax.dot_general(ident, row, (((1,), (1,)), ((), ())),
                           precision=_HI)


def _stable_desc_rank(v_row, v_col, iota_lane, iota_sub):
    m = v_row.shape[1]
    r_mat = jnp.broadcast_to(v_row, (m, m))
    l_mat = jnp.broadcast_to(v_col, (m, m))
    gt = r_mat > l_mat
    tie = (r_mat == l_mat) & (iota_lane < iota_sub)
    return jnp.sum((gt | tie).astype(jnp.float32), axis=1, keepdims=True)


def _onehot_collect(rank_col, data_col, iota_lane_f, m):
    oh = (jnp.broadcast_to(rank_col, (m, m)) == iota_lane_f)
    contrib = jnp.where(oh, jnp.broadcast_to(data_col, (m, m)), 0.0)
    return jnp.sum(contrib, axis=0, keepdims=True)


def _compute_body(res_ref, perm_ref, val_ref, idx_ref, x_ref, out_ref):
    m = _M
    iota_lane = lax.broadcasted_iota(jnp.int32, (m, m), 1)
    iota_sub = lax.broadcasted_iota(jnp.int32, (m, m), 0)
    ident = (iota_lane == iota_sub).astype(jnp.float32)
    iota_lane_f = iota_lane.astype(jnp.float32)

    v_row = res_ref[0]
    v_col = _to_col(v_row, ident)
    rank1 = _stable_desc_rank(v_row, v_col, iota_lane, iota_sub)
    sv_row = _onehot_collect(rank1, v_col, iota_lane_f, m)

    x_mat = x_ref[0]
    vmax = x_ref[0, pl.ds(idx_ref[0, 0, 0], 1), :]
    for k in range(1, _K):
        vmax = jnp.maximum(vmax, x_ref[0, pl.ds(idx_ref[0, 0, k], 1), :])

    s_row = lax.dot_general(vmax, x_mat, (((1,), (1,)), ((), ())),
                            precision=lax.Precision.DEFAULT)
    s_col = _to_col(s_row, ident)

    perm_row = perm_ref[0]
    p_mask = (iota_sub == jnp.broadcast_to(perm_row, (m, m)))
    s_perm = jnp.sum(jnp.where(p_mask, jnp.broadcast_to(s_col, (m, m)), 0.0),
                     axis=0, keepdims=True)

    r_row = (sv_row + s_perm) * 0.5
    r_col = _to_col(r_row, ident)
    rank2 = _stable_desc_rank(r_row, r_col, iota_lane, iota_sub)

    val_col = _to_col(val_ref[0].astype(jnp.float32), ident)
    reord = _onehot_collect(rank2, val_col, iota_lane_f, m)
    out_ref[0] = reord.astype(jnp.int32)


def _assembly_body(reord_ref, ranks_ref, out_ref, *, rows):
    i = pl.program_id(0)

    @pl.when(i == 0)
    def _():
        rf = reord_ref[...].astype(jnp.float32)
        il = lax.broadcasted_iota(jnp.int32, (_Q, _Q), 1)
        isub = lax.broadcasted_iota(jnp.int32, (_Q, _Q), 0)
        ident_q = (il == isub).astype(jnp.float32)
        t = lax.dot_general(rf, ident_q, (((0,), (0,)), ((), ())),
                            precision=_HI)
        out_ref[0:_M, :] = t.astype(jnp.int32)
        out_ref[_M:rows, :] = ranks_ref[_M:rows, :]

    @pl.when(i != 0)
    def _():
        out_ref[...] = ranks_ref[...]


def kernel(ranks, rerank_dba_final, res_top1000_dba, ranks_trans_1000_pre, x_dba):
    q, m, d = _Q, _M, _D

    res3 = res_top1000_dba.reshape(q, 1, m)
    perm3 = ranks_trans_1000_pre.reshape(q, 1, m)
    val3 = rerank_dba_final.reshape(q, 1, m)
    idx_top = ranks_trans_1000_pre[:, :_K].reshape(q, 1, _K)

    reordered = pl.pallas_call(
        _compute_body,
        grid=(q,),
        in_specs=[
            pl.BlockSpec((1, 1, m), lambda i: (i, 0, 0)),
            pl.BlockSpec((1, 1, m), lambda i: (i, 0, 0)),
            pl.BlockSpec((1, 1, m), lambda i: (i, 0, 0)),
            pl.BlockSpec((1, 1, _K), lambda i: (i, 0, 0), memory_space=pltpu.SMEM),
            pl.BlockSpec((1, m, d), lambda i: (i, 0, 0)),
        ],
        out_specs=pl.BlockSpec((1, 1, m), lambda i: (i, 0, 0)),
        out_shape=jax.ShapeDtypeStruct((q, 1, m), jnp.int32),
    )(res3, perm3, val3, idx_top, x_dba)

    rows = 4000
    nblk = _N_DB // rows
    out = pl.pallas_call(
        functools.partial(_assembly_body, rows=rows),
        grid=(nblk,),
        in_specs=[
            pl.BlockSpec((q, m), lambda i: (0, 0)),
            pl.BlockSpec((rows, q), lambda i: (i, 0)),
        ],
        out_specs=pl.BlockSpec((rows, q), lambda i: (i, 0)),
        out_shape=jax.ShapeDtypeStruct((_N_DB, q), jnp.int32),
    )(reordered.reshape(q, m), ranks)
    return out
